# baseline (speedup 1.0000x reference)
import jax
import jax.numpy as jnp
from jax import lax
from jax.experimental import pallas as pl
from jax.experimental.pallas import tpu as pltpu
from jax.experimental.pallas import tpu_sc as plsc

N = 50000
E = 1600000
NB = 32
BW = 1568
NPAD = 50176
EC = E // NB
WP = 4992
NWP = 10
TAILP = EC - NWP * WP
WA = 128
EPAD = E + 192
PRW = 16
NEG = -3.0e38
RB = 3136


def _mesh():
  return plsc.VectorSubcoreMesh(core_axis_name="c", subcore_axis_name="s")


def _partition_body(src_h, dst_h, norm_h, pay_h, offs_h,
                    dwin, swin, nwin, hist, histv, cursor, offsv,
                    owin, owin_t, pchunk, pchunk_t, hist_sh, sem):
  c = lax.axis_index("c")
  s = lax.axis_index("s")
  wid = c * 16 + s
  ebase = wid * EC
  half_base = c * (E // 2)
  iota16 = lax.iota(jnp.int32, 16)
  z16 = jnp.zeros((16,), jnp.int32)

  hist[pl.ds(0, 16)] = z16
  hist[pl.ds(16, 16)] = z16

  def hist_vec(v, carry):
    d16 = dwin[pl.ds(v * 16, 16)]
    b16 = (((d16 >> 5) * 1338) >> 16)
    rank, last = plsc.scan_count(b16)
    base = plsc.load_gather(hist, [b16])
    plsc.store_scatter(hist, [b16], base + rank, mask=last)
    return carry

  def p1_body(w, carry):
    pltpu.sync_copy(dst_h.at[pl.ds(ebase + w * WP, WP)], dwin)
    lax.fori_loop(0, WP // 16, hist_vec, 0)
    return carry
  lax.fori_loop(0, NWP, p1_body, 0)
  pltpu.sync_copy(dst_h.at[pl.ds(ebase + NWP * WP, TAILP)],
                  dwin.at[pl.ds(0, TAILP)])
  lax.fori_loop(0, TAILP // 16, hist_vec, 0)

  pltpu.sync_copy(hist, hist_sh.at[s])
  plsc.subcore_barrier()
  pltpu.sync_copy(hist_sh, histv)

  pre_lo = z16
  pre_hi = z16
  tot_lo = z16
  tot_hi = z16
  for t in range(16):
    hl = histv[t, pl.ds(0, 16)]
    hh = histv[t, pl.ds(16, 16)]
    use = jnp.int32(t) < s
    pre_lo = pre_lo + jnp.where(use, hl, z16)
    pre_hi = pre_hi + jnp.where(use, hh, z16)
    tot_lo = tot_lo + hl
    tot_hi = tot_hi + hh
  cs_lo = plsc.cumsum(tot_lo)
  cs_hi = plsc.cumsum(tot_hi)
  total_lo = cs_lo[15]
  total = total_lo + cs_hi[15]
  ex_lo = cs_lo - tot_lo
  ex_hi = cs_hi - tot_hi + total_lo
  cursor[pl.ds(0, 16)] = half_base + ex_lo + pre_lo
  cursor[pl.ds(16, 16)] = half_base + ex_hi + pre_hi
  offsv[pl.ds(0, 16)] = half_base + ex_lo
  offsv[pl.ds(16, 16)] = half_base + ex_hi
  offsv[pl.ds(32, 16)] = jnp.where(iota16 == 0, half_base + total, 0)

  @pl.when(s == 0)
  def _():
    pltpu.sync_copy(offsv, offs_h.at[c])

  def rank_vec(d16):
    b16 = (((d16 >> 5) * 1338) >> 16)
    rank, last = plsc.scan_count(b16)
    base = plsc.load_gather(cursor, [b16])
    plsc.store_scatter(cursor, [b16], base + rank, mask=last)
    return base + rank - 1

  def stage_vec(vv, src_off, pch, ow):
    base_e = vv * 16
    d16 = dwin[pl.ds(src_off + base_e, 16)]
    s16 = swin[pl.ds(src_off + base_e, 16)]
    n16 = nwin[pl.ds(src_off + base_e, 16)]
    o16 = rank_vec(d16)
    ow[pl.ds(base_e, 16)] = o16
    col = iota16 + base_e
    plsc.store_scatter(pch, [col, z16], s16)
    plsc.store_scatter(pch, [col, z16 + 1], d16)
    plsc.store_scatter(pch, [col, z16 + 2], n16)

  def p2_body(w, carry):
    eb = ebase + w * WP
    pltpu.sync_copy(dst_h.at[pl.ds(eb, WP)], dwin)
    pltpu.sync_copy(src_h.at[pl.ds(eb, WP)], swin)
    pltpu.sync_copy(norm_h.at[pl.ds(eb, WP)], nwin)

    def chunk_body(j, carry2):
      for vv in range(8):
        stage_vec(vv, j * 128, pchunk.at[j], owin.at[j])
      pltpu.async_copy(pchunk.at[j], pay_h.at[owin.at[j]], sem)
      return carry2
    lax.fori_loop(0, 39, chunk_body, 0)

    def drain(j, carry3):
      pltpu.make_async_copy(pchunk.at[j], pay_h.at[owin.at[j]], sem).wait()
      return carry3
    lax.fori_loop(0, 39, drain, 0)
    return carry
  lax.fori_loop(0, NWP, p2_body, 0)

  tb = ebase + NWP * WP
  pltpu.sync_copy(dst_h.at[pl.ds(tb, TAILP)], dwin.at[pl.ds(0, TAILP)])
  pltpu.sync_copy(src_h.at[pl.ds(tb, TAILP)], swin.at[pl.ds(0, TAILP)])
  pltpu.sync_copy(norm_h.at[pl.ds(tb, TAILP)], nwin.at[pl.ds(0, TAILP)])
  for vv in range(TAILP // 16):
    stage_vec(vv, 0, pchunk_t, owin_t)
  pltpu.async_copy(pchunk_t, pay_h.at[owin_t], sem)
  pltpu.make_async_copy(pchunk_t, pay_h.at[owin_t], sem).wait()


def _run_partition(src, dst, norm_i):
  kern = pl.kernel(
      _partition_body,
      out_type=[
          jax.ShapeDtypeStruct((EPAD, PRW), jnp.int32),
          jax.ShapeDtypeStruct((2, 48), jnp.int32),
      ],
      mesh=_mesh(),
      compiler_params=pltpu.CompilerParams(needs_layout_passes=False, use_tc_tiling_on_sc=False),
      scratch_types=[
          pltpu.VMEM((WP,), jnp.int32),
          pltpu.VMEM((WP,), jnp.int32),
          pltpu.VMEM((WP,), jnp.int32),
          pltpu.VMEM((NB,), jnp.int32),
          pltpu.VMEM((16, NB), jnp.int32),
          pltpu.VMEM((NB,), jnp.int32),
          pltpu.VMEM((48,), jnp.int32),
          pltpu.VMEM((39, 128), jnp.int32),
          pltpu.VMEM((TAILP,), jnp.int32),
          pltpu.VMEM((39, 128, PRW), jnp.int32),
          pltpu.VMEM((TAILP, PRW), jnp.int32),
          pltpu.VMEM_SHARED((16, NB), jnp.int32),
          pltpu.SemaphoreType.DMA,
      ],
  )
  return kern(src, dst, norm_i)


def _make_agg_body(Fp, cnt_lane, stage_sp):
  nh = Fp // 16

  def body(pay_h, offs_h, xe_h, aggS_h, aggM_h,
           offsv, pwin, sbuf, dbase, rows, accS, accM, sem, *maybe_sp):
    c = lax.axis_index("c")
    s = lax.axis_index("s")
    b = c * 16 + s
    node_base = b * BW
    iota16 = lax.iota(jnp.int32, 16)
    z16 = jnp.zeros((16,), jnp.int32)

    if stage_sp:
      xe_src = maybe_sp[0]
      @pl.when(s == 0)
      def _():
        pltpu.sync_copy(xe_h, xe_src)
      plsc.subcore_barrier()
    else:
      xe_src = xe_h
    pltpu.sync_copy(offs_h, offsv)

    zf = jnp.zeros((16,), jnp.float32)
    mf = jnp.full((16,), NEG, jnp.float32)
    def init_body(i, carry):
      accS[pl.ds(i * 16, 16)] = zf
      accM[pl.ds(i * 16, 16)] = mf
      return carry
    lax.fori_loop(0, BW * Fp // 16, init_body, 0)

    cnt_masks = [iota16 + hh * 16 == cnt_lane for hh in range(nh)]

    for half in range(2):
      bvec = jnp.full((16,), b, jnp.int32)
      start = plsc.load_gather(offsv, [jnp.full((16,), half, jnp.int32),
                                       bvec])[0]
      end = plsc.load_gather(offsv, [jnp.full((16,), half, jnp.int32),
                                     bvec + 1])[0]
      astart = start & ~7
      nwin = (end - astart + (WA - 1)) >> 7

      def prefetch(w):
        par = w & 1
        wbase = pl.multiple_of(astart + w * WA, 8)
        pltpu.sync_copy(pay_h.at[pl.ds(wbase, WA)], pwin.at[par])
        pv = jnp.full((16,), par, jnp.int32)
        def clamp_body(v, carry2):
          rowv = iota16 + v * 16
          sv = plsc.load_gather(pwin, [pv, rowv, z16])
          sbuf[par, pl.ds(v * 16, 16)] = jnp.clip(sv, 0, N - 1)
          dv = plsc.load_gather(pwin, [pv, rowv, z16 + 1])
          dbase[par, pl.ds(v * 16, 16)] = (dv - node_base) * Fp
          return carry2
        lax.fori_loop(0, WA // 16, clamp_body, 0)
        pltpu.async_copy(xe_src.at[sbuf.at[par]], rows.at[par], sem)

      @pl.when(nwin > 0)
      def _():
        prefetch(0)

      def w_body(w, carry):
        par = w & 1
        pv = jnp.full((16,), par, jnp.int32)
        pltpu.make_async_copy(xe_src.at[sbuf.at[par]], rows.at[par],
                              sem).wait()
        @pl.when(w + 1 < nwin)
        def _():
          prefetch(w + 1)
        wbase = pl.multiple_of(astart + w * WA, 8)
        lo = jnp.maximum(start - wbase, 0)
        cnt = jnp.minimum(jnp.int32(WA), end - wbase)

        def edge_front(e):
          ev = jnp.full((16,), e, jnp.int32)
          nsp = plsc.bitcast(plsc.load_gather(pwin, [pv, ev, z16 + 2]),
                             jnp.float32)
          idx0 = plsc.load_gather(dbase, [pv, ev]) + iota16
          rs = [rows[par, e, pl.ds(hh * 16, 16)] for hh in range(nh)]
          return nsp, idx0, rs

        def quad_body(k, carry2):
          e = lo + k * 8
          es = [e] + [jnp.minimum(e + i, cnt - 1) for i in range(1, 8)]
          msks = [None] + [(z16 + e + i) < cnt for i in range(1, 8)]
          fronts = [edge_front(ei) for ei in es]
          idx0s = [f[1] for f in fronts]
          for hh in range(nh):
            ixs = [idx0s[i] + hh * 16 for i in range(8)]
            curs = [plsc.load_gather(accM, [ixs[i]]) for i in range(8)]
            news = []
            for i in range(8):
              ci = curs[i]
              for jprev in range(i):
                ci = jnp.where(ixs[i] == ixs[jprev], news[jprev], ci)
              news.append(jnp.maximum(ci, fronts[i][2][hh]))
            for i in range(8):
              plsc.store_scatter(accM, [ixs[i]], news[i], mask=msks[i])
          for hh in range(nh):
            for i in range(8):
              mlt = jnp.where(cnt_masks[hh], jnp.float32(1.0), fronts[i][0])
              plsc.addupdate_scatter(accS, [idx0s[i] + hh * 16],
                                     fronts[i][2][hh] * mlt, mask=msks[i])
          return carry2
        lax.fori_loop(0, (cnt - lo + 7) >> 3, quad_body, 0)
        return carry
      lax.fori_loop(0, nwin, w_body, 0)

    obase = pl.multiple_of(node_base * Fp, 8)
    pltpu.sync_copy(accS, aggS_h.at[pl.ds(obase, BW * Fp)])
    pltpu.sync_copy(accM, aggM_h.at[pl.ds(obase, BW * Fp)])

  return body


def _run_agg(pay, offs, xe, Fp, cnt_lane, stage_sp=False):
  kern = pl.kernel(
      _make_agg_body(Fp, cnt_lane, stage_sp),
      out_type=[
          jax.ShapeDtypeStruct((NPAD * Fp,), jnp.float32),
          jax.ShapeDtypeStruct((NPAD * Fp,), jnp.float32),
      ],
      mesh=_mesh(),
      compiler_params=pltpu.CompilerParams(needs_layout_passes=False,
                                           use_tc_tiling_on_sc=False),
      scratch_types=[
          pltpu.VMEM((2, 48), jnp.int32),
          pltpu.VMEM((2, WA, PRW), jnp.int32),
          pltpu.VMEM((2, WA), jnp.int32),
          pltpu.VMEM((2, WA), jnp.int32),
          pltpu.VMEM((2, WA, Fp), jnp.float32),
          pltpu.VMEM((BW * Fp,), jnp.float32),
          pltpu.VMEM((BW * Fp,), jnp.float32),
          pltpu.SemaphoreType.DMA,
      ] + ([pltpu.VMEM_SHARED((NPAD, Fp), jnp.float32)] if stage_sp else []),
  )
  aggS, aggM = kern(pay, offs, xe)
  return aggS.reshape(NPAD, Fp), aggM.reshape(NPAD, Fp)


def _dense1_body(xe1, aggS, aggM, Wl1m, Wr1m, b1m, Wl1x, Wr1x, b1x, xe2):
  xl = xe1[:, 0:5]
  cntv = aggS[:, 5:6]
  aggm = aggS[:, 0:5] / jnp.maximum(cntv, 1.0)
  mraw = aggM[:, 0:5]
  aggx = jnp.where(mraw > -1.0e38, mraw, 0.0)
  y = jnp.maximum(aggm @ Wl1m[...] + xl @ Wr1m[...] + b1m[...], 0.0)
  z = jnp.maximum(aggx @ Wl1x[...] + xl @ Wr1x[...] + b1x[...], 0.0)
  rb = y.shape[0]
  pad = jnp.zeros((rb, 3), jnp.float32)
  one = jnp.ones((rb, 1), jnp.float32)
  xe2[...] = jnp.concatenate([y, z, one, pad], axis=1)


def _dense2_body(xe1, xe2, aggS, aggM, Wl2m, Wr2m, b2m, Wl2x, Wr2x, b2x,
                 W3, b3, W4, b4, W5, b5, out):
  xl = xe1[:, 0:5]
  h = xe2[:, 0:28]
  cntv = aggS[:, 28:29]
  aggm = aggS[:, 0:28] / jnp.maximum(cntv, 1.0)
  mraw = aggM[:, 0:28]
  aggx = jnp.where(mraw > -1.0e38, mraw, 0.0)
  y2 = jnp.maximum(aggm @ Wl2m[...] + h @ Wr2m[...] + b2m[...], 0.0)
  z2 = jnp.maximum(aggx @ Wl2x[...] + h @ Wr2x[...] + b2x[...], 0.0)
  h2 = jnp.concatenate([y2, z2, xl], axis=1)
  h3 = jnp.maximum(h2 @ W3[...] + b3[...], 0.0)
  h4 = jnp.maximum(h3 @ W4[...] + b4[...], 0.0)
  o = h4 @ W5[...] + b5[...]
  out[...] = jnp.broadcast_to(o, (o.shape[0], 8))


def _row_spec(fp):
  return pl.BlockSpec((RB, fp), lambda i: (i, 0))


def _full_spec(shape):
  nd = len(shape)
  return pl.BlockSpec(shape, lambda i: (0,) * nd)


def _run_dense1(xe1, aggS1, aggM1, Wl1m, Wr1m, b1m, Wl1x, Wr1x, b1x):
  ws = [Wl1m, Wr1m, b1m, Wl1x, Wr1x, b1x]
  return pl.pallas_call(
      _dense1_body,
      grid=(NPAD // RB,),
      in_specs=[_row_spec(16), _row_spec(16), _row_spec(16)]
      + [_full_spec(w.shape) for w in ws],
      out_specs=_row_spec(32),
      out_shape=jax.ShapeDtypeStruct((NPAD, 32), jnp.float32),
  )(xe1, aggS1, aggM1, *ws)


def _run_dense2(xe1, xe2, aggS2, aggM2, Wl2m, Wr2m, b2m, Wl2x, Wr2x, b2x,
                W3, b3, W4, b4, W5, b5):
  ws = [Wl2m, Wr2m, b2m, Wl2x, Wr2x, b2x, W3, b3, W4, b4, W5, b5]
  return pl.pallas_call(
      _dense2_body,
      grid=(NPAD // RB,),
      in_specs=[_row_spec(16), _row_spec(32), _row_spec(32), _row_spec(32)]
      + [_full_spec(w.shape) for w in ws],
      out_specs=_row_spec(8),
      out_shape=jax.ShapeDtypeStruct((NPAD, 8), jnp.float32),
  )(xe1, xe2, aggS2, aggM2, *ws)


@jax.jit
def kernel(x, edge_index, norm, Wl1m, Wr1m, b1m, Wl1x, Wr1x, b1x,
           Wl2m, Wr2m, b2m, Wl2x, Wr2x, b2x, W3, b3, W4, b4, W5, b5):
  src = edge_index[0]
  dst = edge_index[1]

  xe1 = jnp.concatenate(
      [x, jnp.ones((N, 1), jnp.float32), jnp.zeros((N, 10), jnp.float32)],
      axis=1)
  xe1 = jnp.pad(xe1, ((0, NPAD - N), (0, 0)))

  norm_i = lax.bitcast_convert_type(norm, jnp.int32)
  pay, offs = _run_partition(src, dst, norm_i)
  aggS1, aggM1 = _run_agg(pay, offs, xe1, 16, 5)
  xe2 = _run_dense1(xe1, aggS1, aggM1, Wl1m, Wr1m, b1m, Wl1x, Wr1x, b1x)
  aggS2, aggM2 = _run_agg(pay, offs, xe2, 32, 28)
  out = _run_dense2(xe1, xe2, aggS2, aggM2, Wl2m, Wr2m, b2m, Wl2x, Wr2x, b2x,
                    W3, b3, W4, b4, W5, b5)
  return out[:N, 0]


# speedup vs baseline: 1.0871x; 1.0871x over previous
import jax
import jax.numpy as jnp
from jax import lax
from jax.experimental import pallas as pl
from jax.experimental.pallas import tpu as pltpu
from jax.experimental.pallas import tpu_sc as plsc

N = 50000
E = 1600000
NB = 32
BW = 1568
NPAD = 50176
EC = E // NB
WP = 4992
NWP = 10
TAILP = EC - NWP * WP
WA = 128
EPAD = E + 192
PRW = 16
NEG = -3.0e38
RB = 3136


def _mesh():
  return plsc.VectorSubcoreMesh(core_axis_name="c", subcore_axis_name="s")


def _partition_body(src_h, dst_h, norm_h, pay_h, offs_h,
                    dwin, swin, nwin, hist, histv, cursor, offsv,
                    owin, owin_t, pchunk, pchunk_t, hist_sh, sem):
  c = lax.axis_index("c")
  s = lax.axis_index("s")
  wid = c * 16 + s
  ebase = wid * EC
  half_base = c * (E // 2)
  iota16 = lax.iota(jnp.int32, 16)
  z16 = jnp.zeros((16,), jnp.int32)

  hist[pl.ds(0, 16)] = z16
  hist[pl.ds(16, 16)] = z16

  def hist_vec(v, carry):
    d16 = dwin[pl.ds(v * 16, 16)]
    b16 = (((d16 >> 5) * 1338) >> 16)
    rank, last = plsc.scan_count(b16)
    base = plsc.load_gather(hist, [b16])
    plsc.store_scatter(hist, [b16], base + rank, mask=last)
    return carry

  def p1_body(w, carry):
    pltpu.sync_copy(dst_h.at[pl.ds(ebase + w * WP, WP)], dwin)
    lax.fori_loop(0, WP // 16, hist_vec, 0)
    return carry
  lax.fori_loop(0, NWP, p1_body, 0)
  pltpu.sync_copy(dst_h.at[pl.ds(ebase + NWP * WP, TAILP)],
                  dwin.at[pl.ds(0, TAILP)])
  lax.fori_loop(0, TAILP // 16, hist_vec, 0)

  pltpu.sync_copy(hist, hist_sh.at[s])
  plsc.subcore_barrier()
  pltpu.sync_copy(hist_sh, histv)

  pre_lo = z16
  pre_hi = z16
  tot_lo = z16
  tot_hi = z16
  for t in range(16):
    hl = histv[t, pl.ds(0, 16)]
    hh = histv[t, pl.ds(16, 16)]
    use = jnp.int32(t) < s
    pre_lo = pre_lo + jnp.where(use, hl, z16)
    pre_hi = pre_hi + jnp.where(use, hh, z16)
    tot_lo = tot_lo + hl
    tot_hi = tot_hi + hh
  cs_lo = plsc.cumsum(tot_lo)
  cs_hi = plsc.cumsum(tot_hi)
  total_lo = cs_lo[15]
  total = total_lo + cs_hi[15]
  ex_lo = cs_lo - tot_lo
  ex_hi = cs_hi - tot_hi + total_lo
  cursor[pl.ds(0, 16)] = half_base + ex_lo + pre_lo
  cursor[pl.ds(16, 16)] = half_base + ex_hi + pre_hi
  offsv[pl.ds(0, 16)] = half_base + ex_lo
  offsv[pl.ds(16, 16)] = half_base + ex_hi
  offsv[pl.ds(32, 16)] = jnp.where(iota16 == 0, half_base + total, 0)

  @pl.when(s == 0)
  def _():
    pltpu.sync_copy(offsv, offs_h.at[c])

  def rank_vec(d16):
    b16 = (((d16 >> 5) * 1338) >> 16)
    rank, last = plsc.scan_count(b16)
    base = plsc.load_gather(cursor, [b16])
    plsc.store_scatter(cursor, [b16], base + rank, mask=last)
    return base + rank - 1

  def stage_vec(vv, src_off, pch, ow):
    base_e = vv * 16
    d16 = dwin[pl.ds(src_off + base_e, 16)]
    s16 = swin[pl.ds(src_off + base_e, 16)]
    n16 = nwin[pl.ds(src_off + base_e, 16)]
    o16 = rank_vec(d16)
    ow[pl.ds(base_e, 16)] = o16
    col = iota16 + base_e
    plsc.store_scatter(pch, [col, z16], s16)
    plsc.store_scatter(pch, [col, z16 + 1], d16)
    plsc.store_scatter(pch, [col, z16 + 2], n16)

  def p2_body(w, carry):
    eb = ebase + w * WP
    pltpu.sync_copy(dst_h.at[pl.ds(eb, WP)], dwin)
    pltpu.sync_copy(src_h.at[pl.ds(eb, WP)], swin)
    pltpu.sync_copy(norm_h.at[pl.ds(eb, WP)], nwin)

    def chunk_body(j, carry2):
      for vv in range(8):
        stage_vec(vv, j * 128, pchunk.at[j], owin.at[j])
      pltpu.async_copy(pchunk.at[j], pay_h.at[owin.at[j]], sem)
      return carry2
    lax.fori_loop(0, 39, chunk_body, 0)

    def drain(j, carry3):
      pltpu.make_async_copy(pchunk.at[j], pay_h.at[owin.at[j]], sem).wait()
      return carry3
    lax.fori_loop(0, 39, drain, 0)
    return carry
  lax.fori_loop(0, NWP, p2_body, 0)

  tb = ebase + NWP * WP
  pltpu.sync_copy(dst_h.at[pl.ds(tb, TAILP)], dwin.at[pl.ds(0, TAILP)])
  pltpu.sync_copy(src_h.at[pl.ds(tb, TAILP)], swin.at[pl.ds(0, TAILP)])
  pltpu.sync_copy(norm_h.at[pl.ds(tb, TAILP)], nwin.at[pl.ds(0, TAILP)])
  for vv in range(TAILP // 16):
    stage_vec(vv, 0, pchunk_t, owin_t)
  pltpu.async_copy(pchunk_t, pay_h.at[owin_t], sem)
  pltpu.make_async_copy(pchunk_t, pay_h.at[owin_t], sem).wait()


def _run_partition(src, dst, norm_i):
  kern = pl.kernel(
      _partition_body,
      out_type=[
          jax.ShapeDtypeStruct((EPAD, PRW), jnp.int32),
          jax.ShapeDtypeStruct((2, 48), jnp.int32),
      ],
      mesh=_mesh(),
      compiler_params=pltpu.CompilerParams(needs_layout_passes=False, use_tc_tiling_on_sc=False),
      scratch_types=[
          pltpu.VMEM((WP,), jnp.int32),
          pltpu.VMEM((WP,), jnp.int32),
          pltpu.VMEM((WP,), jnp.int32),
          pltpu.VMEM((NB,), jnp.int32),
          pltpu.VMEM((16, NB), jnp.int32),
          pltpu.VMEM((NB,), jnp.int32),
          pltpu.VMEM((48,), jnp.int32),
          pltpu.VMEM((39, 128), jnp.int32),
          pltpu.VMEM((TAILP,), jnp.int32),
          pltpu.VMEM((39, 128, PRW), jnp.int32),
          pltpu.VMEM((TAILP, PRW), jnp.int32),
          pltpu.VMEM_SHARED((16, NB), jnp.int32),
          pltpu.SemaphoreType.DMA,
      ],
  )
  return kern(src, dst, norm_i)


def _make_agg_body(Fp, cnt_lane, stage_sp):
  nh = Fp // 16

  def body(pay_h, offs_h, xe_h, aggS_h, aggM_h,
           offsv, pwin, sbuf, dbase, rows, accS, accM, sem, *maybe_sp):
    c = lax.axis_index("c")
    s = lax.axis_index("s")
    b = c * 16 + s
    node_base = b * BW
    iota16 = lax.iota(jnp.int32, 16)
    z16 = jnp.zeros((16,), jnp.int32)

    if stage_sp:
      xe_src = maybe_sp[0]
      @pl.when(s == 0)
      def _():
        pltpu.sync_copy(xe_h, xe_src)
      plsc.subcore_barrier()
    else:
      xe_src = xe_h
    pltpu.sync_copy(offs_h, offsv)

    zf = jnp.zeros((16,), jnp.float32)
    mf = jnp.full((16,), NEG, jnp.float32)
    def init_body(i, carry):
      accS[pl.ds(i * 16, 16)] = zf
      accM[pl.ds(i * 16, 16)] = mf
      return carry
    lax.fori_loop(0, BW * Fp // 16, init_body, 0)

    cnt_masks = [iota16 + hh * 16 == cnt_lane for hh in range(nh)]

    for half in range(2):
      bvec = jnp.full((16,), b, jnp.int32)
      start = plsc.load_gather(offsv, [jnp.full((16,), half, jnp.int32),
                                       bvec])[0]
      end = plsc.load_gather(offsv, [jnp.full((16,), half, jnp.int32),
                                     bvec + 1])[0]
      astart = start & ~7
      nwin = (end - astart + (WA - 1)) >> 7

      def prefetch(w):
        par = w & 1
        wbase = pl.multiple_of(astart + w * WA, 8)
        pltpu.sync_copy(pay_h.at[pl.ds(wbase, WA)], pwin.at[par])
        pv = jnp.full((16,), par, jnp.int32)
        def clamp_body(v, carry2):
          rowv = iota16 + v * 16
          sv = plsc.load_gather(pwin, [pv, rowv, z16])
          sbuf[par, pl.ds(v * 16, 16)] = jnp.clip(sv, 0, N - 1)
          dv = plsc.load_gather(pwin, [pv, rowv, z16 + 1])
          dbase[par, pl.ds(v * 16, 16)] = (dv - node_base) * Fp
          return carry2
        lax.fori_loop(0, WA // 16, clamp_body, 0)
        pltpu.async_copy(xe_src.at[sbuf.at[par]], rows.at[par], sem)

      @pl.when(nwin > 0)
      def _():
        prefetch(0)

      def w_body(w, carry):
        par = w & 1
        pv = jnp.full((16,), par, jnp.int32)
        pltpu.make_async_copy(xe_src.at[sbuf.at[par]], rows.at[par],
                              sem).wait()
        @pl.when(w + 1 < nwin)
        def _():
          prefetch(w + 1)
        wbase = pl.multiple_of(astart + w * WA, 8)
        lo = jnp.maximum(start - wbase, 0)
        cnt = jnp.minimum(jnp.int32(WA), end - wbase)

        def edge_front(e):
          ev = jnp.full((16,), e, jnp.int32)
          nsp = plsc.bitcast(plsc.load_gather(pwin, [pv, ev, z16 + 2]),
                             jnp.float32)
          idx0 = plsc.load_gather(dbase, [pv, ev]) + iota16
          rs = [rows[par, e, pl.ds(hh * 16, 16)] for hh in range(nh)]
          return nsp, idx0, rs

        def quad_body(k, carry2):
          e = lo + k * 4
          es = [e] + [jnp.minimum(e + i, cnt - 1) for i in range(1, 4)]
          msks = [None] + [(z16 + e + i) < cnt for i in range(1, 4)]
          fronts = [edge_front(ei) for ei in es]
          idx0s = [f[1] for f in fronts]
          for hh in range(nh):
            ixs = [idx0s[i] + hh * 16 for i in range(4)]
            curs = [plsc.load_gather(accM, [ixs[i]]) for i in range(4)]
            news = []
            for i in range(4):
              ci = curs[i]
              for jprev in range(i):
                ci = jnp.where(ixs[i] == ixs[jprev], news[jprev], ci)
              news.append(jnp.maximum(ci, fronts[i][2][hh]))
            for i in range(4):
              plsc.store_scatter(accM, [ixs[i]], news[i], mask=msks[i])
          for hh in range(nh):
            for i in range(4):
              mlt = jnp.where(cnt_masks[hh], jnp.float32(1.0), fronts[i][0])
              plsc.addupdate_scatter(accS, [idx0s[i] + hh * 16],
                                     fronts[i][2][hh] * mlt, mask=msks[i])
          return carry2
        lax.fori_loop(0, (cnt - lo + 3) >> 2, quad_body, 0)
        return carry
      lax.fori_loop(0, nwin, w_body, 0)

    obase = pl.multiple_of(node_base * Fp, 8)
    pltpu.sync_copy(accS, aggS_h.at[pl.ds(obase, BW * Fp)])
    pltpu.sync_copy(accM, aggM_h.at[pl.ds(obase, BW * Fp)])

  return body


def _run_agg(pay, offs, xe, Fp, cnt_lane, stage_sp=False):
  kern = pl.kernel(
      _make_agg_body(Fp, cnt_lane, stage_sp),
      out_type=[
          jax.ShapeDtypeStruct((NPAD * Fp,), jnp.float32),
          jax.ShapeDtypeStruct((NPAD * Fp,), jnp.float32),
      ],
      mesh=_mesh(),
      compiler_params=pltpu.CompilerParams(needs_layout_passes=False,
                                           use_tc_tiling_on_sc=False),
      scratch_types=[
          pltpu.VMEM((2, 48), jnp.int32),
          pltpu.VMEM((2, WA, PRW), jnp.int32),
          pltpu.VMEM((2, WA), jnp.int32),
          pltpu.VMEM((2, WA), jnp.int32),
          pltpu.VMEM((2, WA, Fp), jnp.float32),
          pltpu.VMEM((BW * Fp,), jnp.float32),
          pltpu.VMEM((BW * Fp,), jnp.float32),
          pltpu.SemaphoreType.DMA,
      ] + ([pltpu.VMEM_SHARED((NPAD, Fp), jnp.float32)] if stage_sp else []),
  )
  aggS, aggM = kern(pay, offs, xe)
  return aggS.reshape(NPAD, Fp), aggM.reshape(NPAD, Fp)


def _dense1_body(xe1, aggS, aggM, Wl1m, Wr1m, b1m, Wl1x, Wr1x, b1x, xe2):
  xl = xe1[:, 0:5]
  cntv = aggS[:, 5:6]
  aggm = aggS[:, 0:5] / jnp.maximum(cntv, 1.0)
  mraw = aggM[:, 0:5]
  aggx = jnp.where(mraw > -1.0e38, mraw, 0.0)
  y = jnp.maximum(aggm @ Wl1m[...] + xl @ Wr1m[...] + b1m[...], 0.0)
  z = jnp.maximum(aggx @ Wl1x[...] + xl @ Wr1x[...] + b1x[...], 0.0)
  rb = y.shape[0]
  pad = jnp.zeros((rb, 3), jnp.float32)
  one = jnp.ones((rb, 1), jnp.float32)
  xe2[...] = jnp.concatenate([y, z, one, pad], axis=1)


def _dense2_body(xe1, xe2, aggS, aggM, Wl2m, Wr2m, b2m, Wl2x, Wr2x, b2x,
                 W3, b3, W4, b4, W5, b5, out):
  xl = xe1[:, 0:5]
  h = xe2[:, 0:28]
  cntv = aggS[:, 28:29]
  aggm = aggS[:, 0:28] / jnp.maximum(cntv, 1.0)
  mraw = aggM[:, 0:28]
  aggx = jnp.where(mraw > -1.0e38, mraw, 0.0)
  y2 = jnp.maximum(aggm @ Wl2m[...] + h @ Wr2m[...] + b2m[...], 0.0)
  z2 = jnp.maximum(aggx @ Wl2x[...] + h @ Wr2x[...] + b2x[...], 0.0)
  h2 = jnp.concatenate([y2, z2, xl], axis=1)
  h3 = jnp.maximum(h2 @ W3[...] + b3[...], 0.0)
  h4 = jnp.maximum(h3 @ W4[...] + b4[...], 0.0)
  o = h4 @ W5[...] + b5[...]
  out[...] = jnp.broadcast_to(o, (o.shape[0], 8))


def _row_spec(fp):
  return pl.BlockSpec((RB, fp), lambda i: (i, 0))


def _full_spec(shape):
  nd = len(shape)
  return pl.BlockSpec(shape, lambda i: (0,) * nd)


def _run_dense1(xe1, aggS1, aggM1, Wl1m, Wr1m, b1m, Wl1x, Wr1x, b1x):
  ws = [Wl1m, Wr1m, b1m, Wl1x, Wr1x, b1x]
  return pl.pallas_call(
      _dense1_body,
      grid=(NPAD // RB,),
      in_specs=[_row_spec(16), _row_spec(16), _row_spec(16)]
      + [_full_spec(w.shape) for w in ws],
      out_specs=_row_spec(32),
      out_shape=jax.ShapeDtypeStruct((NPAD, 32), jnp.float32),
  )(xe1, aggS1, aggM1, *ws)


def _run_dense2(xe1, xe2, aggS2, aggM2, Wl2m, Wr2m, b2m, Wl2x, Wr2x, b2x,
                W3, b3, W4, b4, W5, b5):
  ws = [Wl2m, Wr2m, b2m, Wl2x, Wr2x, b2x, W3, b3, W4, b4, W5, b5]
  return pl.pallas_call(
      _dense2_body,
      grid=(NPAD // RB,),
      in_specs=[_row_spec(16), _row_spec(32), _row_spec(32), _row_spec(32)]
      + [_full_spec(w.shape) for w in ws],
      out_specs=_row_spec(8),
      out_shape=jax.ShapeDtypeStruct((NPAD, 8), jnp.float32),
  )(xe1, xe2, aggS2, aggM2, *ws)


@jax.jit
def kernel(x, edge_index, norm, Wl1m, Wr1m, b1m, Wl1x, Wr1x, b1x,
           Wl2m, Wr2m, b2m, Wl2x, Wr2x, b2x, W3, b3, W4, b4, W5, b5):
  src = edge_index[0]
  dst = edge_index[1]

  xe1 = jnp.concatenate(
      [x, jnp.ones((N, 1), jnp.float32), jnp.zeros((N, 10), jnp.float32)],
      axis=1)
  xe1 = jnp.pad(xe1, ((0, NPAD - N), (0, 0)))

  norm_i = lax.bitcast_convert_type(norm, jnp.int32)
  pay, offs = _run_partition(src, dst, norm_i)
  aggS1, aggM1 = _run_agg(pay, offs, xe1, 16, 5)
  xe2 = _run_dense1(xe1, aggS1, aggM1, Wl1m, Wr1m, b1m, Wl1x, Wr1x, b1x)
  aggS2, aggM2 = _run_agg(pay, offs, xe2, 32, 28)
  out = _run_dense2(xe1, xe2, aggS2, aggM2, Wl2m, Wr2m, b2m, Wl2x, Wr2x, b2x,
                    W3, b3, W4, b4, W5, b5)
  return out[:N, 0]
